# Initial kernel scaffold; baseline (speedup 1.0000x reference)
import jax
import jax.numpy as jnp
from jax.experimental import pallas as pl


def kernel(x, edge_index, W0, b0, gamma, beta, W1, b1):
    raise NotImplementedError("write your pallas kernel here")


# speedup vs baseline: 16.2890x; 16.2890x over previous
import functools

import jax
import jax.numpy as jnp
from jax import lax
from jax.experimental import pallas as pl
from jax.experimental.pallas import tpu as pltpu
from jax.experimental.pallas import tpu_sc as plsc

N = 10000
D = 128
E = 320000
NC = 2
NS = 16
NW = NC * NS
EPW = E // NW
C = 80
CH = EPW // C

NP_DEG = 10240
DEG_CH = NP_DEG // NS

ROW_CH = 200
N_ROW_CH = N // ROW_CH


def _sc_mesh():
    return plsc.VectorSubcoreMesh(core_axis_name="c", subcore_axis_name="s")


def _sc_deg(dst_r):
    @functools.partial(
        pl.kernel,
        mesh=_sc_mesh(),
        out_type=jax.ShapeDtypeStruct((NC * NP_DEG,), jnp.float32),
        scratch_types=[
            pltpu.VMEM((CH, C), jnp.int32),
            pltpu.VMEM((C,), jnp.float32),
            pltpu.VMEM((DEG_CH,), jnp.float32),
            pltpu.VMEM_SHARED((NP_DEG,), jnp.float32),
        ],
    )
    def k(dst_hbm, out_hbm, idx_v, ones_v, zer_v, acc_sh):
        c = lax.axis_index("c")
        s = lax.axis_index("s")
        w = c * NS + s
        for i in range(C // 16):
            ones_v[pl.ds(i * 16, 16)] = jnp.ones((16,), jnp.float32)
        for i in range(DEG_CH // 16):
            zer_v[pl.ds(i * 16, 16)] = jnp.zeros((16,), jnp.float32)
        pltpu.sync_copy(zer_v, acc_sh.at[pl.ds(s * DEG_CH, DEG_CH)])
        pltpu.sync_copy(dst_hbm.at[w], idx_v)
        plsc.subcore_barrier()

        def step(i, carry):
            pltpu.sync_copy(ones_v, acc_sh.at[idx_v.at[i]], add=True)
            return carry

        lax.fori_loop(0, CH, step, 0)
        plsc.subcore_barrier()
        pltpu.sync_copy(
            acc_sh.at[pl.ds(s * DEG_CH, DEG_CH)],
            out_hbm.at[pl.ds(c * NP_DEG + s * DEG_CH, DEG_CH)],
        )

    return k(dst_r)


def _sc_msg(src_r, dst_r, h):
    @functools.partial(
        pl.kernel,
        mesh=_sc_mesh(),
        out_type=jax.ShapeDtypeStruct((NC, N, D), jnp.float32),
        scratch_types=[
            pltpu.VMEM((CH, C), jnp.int32),
            pltpu.VMEM((C,), jnp.int32),
            pltpu.VMEM((C, D), jnp.float32),
            pltpu.VMEM_SHARED((N, D), jnp.float32),
            pltpu.SemaphoreType.DMA,
        ],
    )
    def k(src_hbm, dst_hbm, h_hbm, out_hbm, src_v, dst_c, rows_v, acc_sh, sem):
        c = lax.axis_index("c")
        s = lax.axis_index("s")
        w = c * NS + s
        for j in range(4):
            q = s + j * NS

            @pl.when(q < N_ROW_CH)
            def _():
                r0 = q * ROW_CH
                pltpu.sync_copy(
                    h_hbm.at[pl.ds(r0, ROW_CH)], acc_sh.at[pl.ds(r0, ROW_CH)]
                )

        pltpu.sync_copy(src_hbm.at[w], src_v)
        plsc.subcore_barrier()

        def step(i, carry):
            pltpu.sync_copy(dst_hbm.at[w, i], dst_c)
            pltpu.async_copy(h_hbm.at[src_v.at[i]], rows_v, sem).wait()
            pltpu.sync_copy(rows_v, acc_sh.at[dst_c], add=True)
            return carry

        lax.fori_loop(0, CH, step, 0)
        plsc.subcore_barrier()
        for j in range(4):
            q = s + j * NS

            @pl.when(q < N_ROW_CH)
            def _():
                r0 = q * ROW_CH
                pltpu.sync_copy(
                    acc_sh.at[pl.ds(r0, ROW_CH)], out_hbm.at[c, pl.ds(r0, ROW_CH)]
                )

    return k(src_r, dst_r, h)


BR = 1000


def _dis(deg_ref):
    return lax.rsqrt(deg_ref[:, 0:1] + deg_ref[:, 1:2] + 1.0)


def _tc1_body(x_ref, w_ref, deg_ref, out_ref):
    mm = jnp.dot(x_ref[...], w_ref[...], preferred_element_type=jnp.float32)
    out_ref[...] = mm * _dis(deg_ref)


def _tc1(x, W0, degT):
    return pl.pallas_call(
        _tc1_body,
        grid=(N // BR,),
        in_specs=[
            pl.BlockSpec((BR, D), lambda i: (i, 0)),
            pl.BlockSpec((D, D), lambda i: (0, 0)),
            pl.BlockSpec((BR, 2), lambda i: (i, 0)),
        ],
        out_specs=pl.BlockSpec((BR, D), lambda i: (i, 0)),
        out_shape=jax.ShapeDtypeStruct((N, D), jnp.float32),
    )(x, W0, degT)


def _tc2_body(p_ref, h_ref, deg_ref, b0_ref, g_ref, be_ref, w1_ref, out_ref):
    dis = _dis(deg_ref)
    t = (p_ref[0] + p_ref[1] - h_ref[...]) * dis + b0_ref[...]
    mu = jnp.mean(t, axis=1, keepdims=True)
    var = jnp.mean((t - mu) ** 2, axis=1, keepdims=True)
    tn = (t - mu) * lax.rsqrt(var + 1e-5) * g_ref[...] + be_ref[...]
    tr = jnp.maximum(tn, 0.0)
    out_ref[...] = (
        jnp.dot(tr, w1_ref[...], preferred_element_type=jnp.float32) * dis
    )


def _tc2(p, h0p, degT, b0, gamma, beta, W1):
    return pl.pallas_call(
        _tc2_body,
        grid=(N // BR,),
        in_specs=[
            pl.BlockSpec((NC, BR, D), lambda i: (0, i, 0)),
            pl.BlockSpec((BR, D), lambda i: (i, 0)),
            pl.BlockSpec((BR, 2), lambda i: (i, 0)),
            pl.BlockSpec((1, D), lambda i: (0, 0)),
            pl.BlockSpec((1, D), lambda i: (0, 0)),
            pl.BlockSpec((1, D), lambda i: (0, 0)),
            pl.BlockSpec((D, D), lambda i: (0, 0)),
        ],
        out_specs=pl.BlockSpec((BR, D), lambda i: (i, 0)),
        out_shape=jax.ShapeDtypeStruct((N, D), jnp.float32),
    )(p, h0p, degT, b0, gamma, beta, W1)


def _tc3_body(p_ref, h_ref, deg_ref, b1_ref, out_ref):
    out_ref[...] = (p_ref[0] + p_ref[1] - h_ref[...]) * _dis(deg_ref) + b1_ref[...]


def _tc3(p, h1p, degT, b1):
    return pl.pallas_call(
        _tc3_body,
        grid=(N // BR,),
        in_specs=[
            pl.BlockSpec((NC, BR, D), lambda i: (0, i, 0)),
            pl.BlockSpec((BR, D), lambda i: (i, 0)),
            pl.BlockSpec((BR, 2), lambda i: (i, 0)),
            pl.BlockSpec((1, D), lambda i: (0, 0)),
        ],
        out_specs=pl.BlockSpec((BR, D), lambda i: (i, 0)),
        out_shape=jax.ShapeDtypeStruct((N, D), jnp.float32),
    )(p, h1p, degT, b1)


def kernel(x, edge_index, W0, b0, gamma, beta, W1, b1):
    ei = edge_index.astype(jnp.int32)
    src_r = ei[0].reshape(NW, CH, C)
    dst_r = ei[1].reshape(NW, CH, C)
    b0r = b0.reshape(1, D)
    b1r = b1.reshape(1, D)
    gr = gamma.reshape(1, D)
    ber = beta.reshape(1, D)

    degf = _sc_deg(dst_r)
    degT = jnp.stack([degf[:N], degf[NP_DEG : NP_DEG + N]], axis=1)
    h0p = _tc1(x, W0, degT)
    p0 = _sc_msg(src_r, dst_r, h0p)
    h1p = _tc2(p0, h0p, degT, b0r, gr, ber, W1)
    p1 = _sc_msg(src_r, dst_r, h1p)
    out = _tc3(p1, h1p, degT, b1r)
    return out


# speedup vs baseline: 30.7690x; 1.8889x over previous
import functools

import jax
import jax.numpy as jnp
from jax import lax
from jax.experimental import pallas as pl
from jax.experimental.pallas import tpu as pltpu
from jax.experimental.pallas import tpu_sc as plsc

N = 10000
D = 128
E = 320000
NC = 2
NS = 16
NW = NC * NS
EPW = E // NW
C = 80
CH = EPW // C

NP_DEG = 10240
DEG_CH = NP_DEG // NS

ROW_CH = 200
N_ROW_CH = N // ROW_CH


def _sc_mesh():
    return plsc.VectorSubcoreMesh(core_axis_name="c", subcore_axis_name="s")


def _sc_deg(dst_r):
    @functools.partial(
        pl.kernel,
        mesh=_sc_mesh(),
        out_type=jax.ShapeDtypeStruct((NC * NP_DEG,), jnp.float32),
        scratch_types=[
            pltpu.VMEM((CH, C), jnp.int32),
            pltpu.VMEM((C,), jnp.float32),
            pltpu.VMEM((DEG_CH,), jnp.float32),
            pltpu.VMEM_SHARED((NP_DEG,), jnp.float32),
        ],
    )
    def k(dst_hbm, out_hbm, idx_v, ones_v, zer_v, acc_sh):
        c = lax.axis_index("c")
        s = lax.axis_index("s")
        w = c * NS + s
        for i in range(C // 16):
            ones_v[pl.ds(i * 16, 16)] = jnp.ones((16,), jnp.float32)
        for i in range(DEG_CH // 16):
            zer_v[pl.ds(i * 16, 16)] = jnp.zeros((16,), jnp.float32)
        pltpu.sync_copy(zer_v, acc_sh.at[pl.ds(s * DEG_CH, DEG_CH)])
        pltpu.sync_copy(dst_hbm.at[w], idx_v)
        plsc.subcore_barrier()

        def step(i, carry):
            pltpu.sync_copy(ones_v, acc_sh.at[idx_v.at[i]], add=True)
            return carry

        lax.fori_loop(0, CH, step, 0)
        plsc.subcore_barrier()
        pltpu.sync_copy(
            acc_sh.at[pl.ds(s * DEG_CH, DEG_CH)],
            out_hbm.at[pl.ds(c * NP_DEG + s * DEG_CH, DEG_CH)],
        )

    return k(dst_r)


def _sc_msg(src_r, dst_r, h):
    @functools.partial(
        pl.kernel,
        mesh=_sc_mesh(),
        out_type=jax.ShapeDtypeStruct((NC, N, D), jnp.float32),
        scratch_types=[
            pltpu.VMEM((CH, C), jnp.int32),
            pltpu.VMEM((C,), jnp.int32),
            pltpu.VMEM((C,), jnp.int32),
            pltpu.VMEM((C, D), jnp.float32),
            pltpu.VMEM((C, D), jnp.float32),
            pltpu.VMEM_SHARED((N, D), jnp.float32),
            pltpu.SemaphoreType.DMA,
            pltpu.SemaphoreType.DMA,
            pltpu.SemaphoreType.DMA,
            pltpu.SemaphoreType.DMA,
        ],
    )
    def k(src_hbm, dst_hbm, h_hbm, out_hbm, src_v, dst_c0, dst_c1, rows0, rows1,
          acc_sh, semi0, semi1, semg0, semg1):
        c = lax.axis_index("c")
        s = lax.axis_index("s")
        w = c * NS + s
        for j in range(4):
            q = s + j * NS

            @pl.when(q < N_ROW_CH)
            def _():
                r0 = q * ROW_CH
                pltpu.sync_copy(
                    h_hbm.at[pl.ds(r0, ROW_CH)], acc_sh.at[pl.ds(r0, ROW_CH)]
                )

        pltpu.sync_copy(src_hbm.at[w], src_v)
        plsc.subcore_barrier()

        def _start(i, dstb, rowsb, semi, semg):
            pltpu.async_copy(dst_hbm.at[w, i, 0], dstb, semi)
            pltpu.async_copy(h_hbm.at[src_v.at[i]], rowsb, semg)

        def _finish(i, dstb, rowsb, semi, semg):
            pltpu.make_async_copy(dst_hbm.at[w, i, 0], dstb, semi).wait()
            pltpu.make_async_copy(h_hbm.at[src_v.at[i]], rowsb, semg).wait()
            pltpu.sync_copy(rowsb, acc_sh.at[dstb], add=True)

        _start(0, dst_c0, rows0, semi0, semg0)

        def step(j, carry):
            i0 = 2 * j
            _start(i0 + 1, dst_c1, rows1, semi1, semg1)
            _finish(i0, dst_c0, rows0, semi0, semg0)
            _start(i0 + 2, dst_c0, rows0, semi0, semg0)
            _finish(i0 + 1, dst_c1, rows1, semi1, semg1)
            return carry

        lax.fori_loop(0, CH // 2, step, 0)
        _finish(CH - 1, dst_c0, rows0, semi0, semg0)
        plsc.subcore_barrier()
        for j in range(4):
            q = s + j * NS

            @pl.when(q < N_ROW_CH)
            def _():
                r0 = q * ROW_CH
                pltpu.sync_copy(
                    acc_sh.at[pl.ds(r0, ROW_CH)], out_hbm.at[c, pl.ds(r0, ROW_CH)]
                )

    return k(src_r, dst_r, h)


BR = 1000


def _dis(deg_ref):
    return lax.rsqrt(deg_ref[:, 0:1] + deg_ref[:, 1:2] + 1.0)


def _tc1_body(x_ref, w_ref, deg_ref, out_ref):
    mm = jnp.dot(x_ref[...], w_ref[...], preferred_element_type=jnp.float32)
    out_ref[...] = mm * _dis(deg_ref)


def _tc1(x, W0, degT):
    return pl.pallas_call(
        _tc1_body,
        grid=(N // BR,),
        in_specs=[
            pl.BlockSpec((BR, D), lambda i: (i, 0)),
            pl.BlockSpec((D, D), lambda i: (0, 0)),
            pl.BlockSpec((BR, 2), lambda i: (i, 0)),
        ],
        out_specs=pl.BlockSpec((BR, D), lambda i: (i, 0)),
        out_shape=jax.ShapeDtypeStruct((N, D), jnp.float32),
    )(x, W0, degT)


def _tc2_body(p_ref, h_ref, deg_ref, b0_ref, g_ref, be_ref, w1_ref, out_ref):
    dis = _dis(deg_ref)
    t = (p_ref[0] + p_ref[1] - h_ref[...]) * dis + b0_ref[...]
    mu = jnp.mean(t, axis=1, keepdims=True)
    var = jnp.mean((t - mu) ** 2, axis=1, keepdims=True)
    tn = (t - mu) * lax.rsqrt(var + 1e-5) * g_ref[...] + be_ref[...]
    tr = jnp.maximum(tn, 0.0)
    out_ref[...] = (
        jnp.dot(tr, w1_ref[...], preferred_element_type=jnp.float32) * dis
    )


def _tc2(p, h0p, degT, b0, gamma, beta, W1):
    return pl.pallas_call(
        _tc2_body,
        grid=(N // BR,),
        in_specs=[
            pl.BlockSpec((NC, BR, D), lambda i: (0, i, 0)),
            pl.BlockSpec((BR, D), lambda i: (i, 0)),
            pl.BlockSpec((BR, 2), lambda i: (i, 0)),
            pl.BlockSpec((1, D), lambda i: (0, 0)),
            pl.BlockSpec((1, D), lambda i: (0, 0)),
            pl.BlockSpec((1, D), lambda i: (0, 0)),
            pl.BlockSpec((D, D), lambda i: (0, 0)),
        ],
        out_specs=pl.BlockSpec((BR, D), lambda i: (i, 0)),
        out_shape=jax.ShapeDtypeStruct((N, D), jnp.float32),
    )(p, h0p, degT, b0, gamma, beta, W1)


def _tc3_body(p_ref, h_ref, deg_ref, b1_ref, out_ref):
    out_ref[...] = (p_ref[0] + p_ref[1] - h_ref[...]) * _dis(deg_ref) + b1_ref[...]


def _tc3(p, h1p, degT, b1):
    return pl.pallas_call(
        _tc3_body,
        grid=(N // BR,),
        in_specs=[
            pl.BlockSpec((NC, BR, D), lambda i: (0, i, 0)),
            pl.BlockSpec((BR, D), lambda i: (i, 0)),
            pl.BlockSpec((BR, 2), lambda i: (i, 0)),
            pl.BlockSpec((1, D), lambda i: (0, 0)),
        ],
        out_specs=pl.BlockSpec((BR, D), lambda i: (i, 0)),
        out_shape=jax.ShapeDtypeStruct((N, D), jnp.float32),
    )(p, h1p, degT, b1)


def kernel(x, edge_index, W0, b0, gamma, beta, W1, b1):
    ei = edge_index.astype(jnp.int32)
    src_r = ei[0].reshape(NW, CH, C)
    dst_r = ei[1].reshape(NW, CH, C)
    dst_r4 = ei[1].reshape(NW, CH, 1, C)
    b0r = b0.reshape(1, D)
    b1r = b1.reshape(1, D)
    gr = gamma.reshape(1, D)
    ber = beta.reshape(1, D)

    degf = _sc_deg(dst_r)
    degT = jnp.stack([degf[:N], degf[NP_DEG : NP_DEG + N]], axis=1)
    h0p = _tc1(x, W0, degT)
    p0 = _sc_msg(src_r, dst_r4, h0p)
    h1p = _tc2(p0, h0p, degT, b0r, gr, ber, W1)
    p1 = _sc_msg(src_r, dst_r4, h1p)
    out = _tc3(p1, h1p, degT, b1r)
    return out


# speedup vs baseline: 35.4703x; 1.1528x over previous
import functools

import jax
import jax.numpy as jnp
from jax import lax
from jax.experimental import pallas as pl
from jax.experimental.pallas import tpu as pltpu
from jax.experimental.pallas import tpu_sc as plsc

N = 10000
D = 128
E = 320000
NC = 2
NS = 16
NW = NC * NS
EPW = E // NW
C = 80
CH = EPW // C

NP_DEG = 10240
DEG_CH = NP_DEG // NS

ROW_CH = 200
N_ROW_CH = N // ROW_CH


def _sc_mesh():
    return plsc.VectorSubcoreMesh(core_axis_name="c", subcore_axis_name="s")


def _sc_deg(dst_r):
    @functools.partial(
        pl.kernel,
        mesh=_sc_mesh(),
        out_type=jax.ShapeDtypeStruct((NC * NP_DEG,), jnp.float32),
        scratch_types=[
            pltpu.VMEM((CH, C), jnp.int32),
            pltpu.VMEM((C,), jnp.float32),
            pltpu.VMEM((DEG_CH,), jnp.float32),
            pltpu.VMEM_SHARED((NP_DEG,), jnp.float32),
        ],
    )
    def k(dst_hbm, out_hbm, idx_v, ones_v, zer_v, acc_sh):
        c = lax.axis_index("c")
        s = lax.axis_index("s")
        w = c * NS + s
        for i in range(C // 16):
            ones_v[pl.ds(i * 16, 16)] = jnp.ones((16,), jnp.float32)
        for i in range(DEG_CH // 16):
            zer_v[pl.ds(i * 16, 16)] = jnp.zeros((16,), jnp.float32)
        pltpu.sync_copy(zer_v, acc_sh.at[pl.ds(s * DEG_CH, DEG_CH)])
        pltpu.sync_copy(dst_hbm.at[w], idx_v)
        plsc.subcore_barrier()

        def step(i, carry):
            pltpu.sync_copy(ones_v, acc_sh.at[idx_v.at[i]], add=True)
            return carry

        lax.fori_loop(0, CH, step, 0)
        plsc.subcore_barrier()
        pltpu.sync_copy(
            acc_sh.at[pl.ds(s * DEG_CH, DEG_CH)],
            out_hbm.at[pl.ds(c * NP_DEG + s * DEG_CH, DEG_CH)],
        )

    return k(dst_r)


def _sc_msg(src_r, dst_r, h):
    @functools.partial(
        pl.kernel,
        mesh=_sc_mesh(),
        out_type=jax.ShapeDtypeStruct((NC, N, D), jnp.float32),
        scratch_types=[
            pltpu.VMEM((CH, C), jnp.int32),
            pltpu.VMEM((C,), jnp.int32),
            pltpu.VMEM((C,), jnp.int32),
            pltpu.VMEM((C,), jnp.int32),
            pltpu.VMEM((C, D), jnp.float32),
            pltpu.VMEM((C, D), jnp.float32),
            pltpu.VMEM((C, D), jnp.float32),
            pltpu.VMEM_SHARED((N, D), jnp.float32),
            pltpu.SemaphoreType.DMA,
            pltpu.SemaphoreType.DMA,
            pltpu.SemaphoreType.DMA,
            pltpu.SemaphoreType.DMA,
            pltpu.SemaphoreType.DMA,
            pltpu.SemaphoreType.DMA,
            pltpu.SemaphoreType.DMA,
            pltpu.SemaphoreType.DMA,
            pltpu.SemaphoreType.DMA,
        ],
    )
    def k(src_hbm, dst_hbm, h_hbm, out_hbm, src_v, dst_c0, dst_c1, dst_c2,
          rows0, rows1, rows2, acc_sh,
          semi0, semi1, semi2, semg0, semg1, semg2, sems0, sems1, sems2):
        c = lax.axis_index("c")
        s = lax.axis_index("s")
        w = c * NS + s
        for j in range(4):
            q = s + j * NS

            @pl.when(q < N_ROW_CH)
            def _():
                r0 = q * ROW_CH
                pltpu.sync_copy(
                    h_hbm.at[pl.ds(r0, ROW_CH)], acc_sh.at[pl.ds(r0, ROW_CH)]
                )

        pltpu.sync_copy(src_hbm.at[w], src_v)
        plsc.subcore_barrier()

        bufs = (
            (dst_c0, rows0, semi0, semg0, sems0),
            (dst_c1, rows1, semi1, semg1, sems1),
            (dst_c2, rows2, semi2, semg2, sems2),
        )

        def _start(i, b):
            dstb, rowsb, semi, semg, _ = bufs[b]
            pltpu.async_copy(dst_hbm.at[w, i, 0], dstb, semi)
            pltpu.async_copy(h_hbm.at[src_v.at[i]], rowsb, semg)

        def _wait_scat(b):
            dstb, rowsb, _, _, sems = bufs[b]
            pltpu.make_async_copy(rowsb, acc_sh.at[dstb], sems).wait()

        def _step(i, b, first, last):
            dstb, rowsb, semi, semg, sems = bufs[b]
            if not first:

                @pl.when(i >= 1)
                def _():
                    _wait_scat((b + 2) % 3)

            if not last:

                @pl.when(i + 2 < CH)
                def _():
                    _start(i + 2, (b + 2) % 3)

            pltpu.make_async_copy(dst_hbm.at[w, i, 0], dstb, semi).wait()
            pltpu.make_async_copy(h_hbm.at[src_v.at[i]], rowsb, semg).wait()
            pltpu.async_copy(rowsb, acc_sh.at[dstb], sems, add=True)

        _start(0, 0)
        _start(1, 1)
        _step(0, 0, True, False)

        def step(j, carry):
            i0 = 3 * j + 1
            _step(i0, 1, False, False)
            _step(i0 + 1, 2, False, False)
            _step(i0 + 2, 0, False, False)
            return carry

        lax.fori_loop(0, (CH - 2) // 3, step, 0)
        _step(CH - 1, (CH - 1) % 3, False, True)
        _wait_scat((CH - 1) % 3)
        plsc.subcore_barrier()
        for j in range(4):
            q = s + j * NS

            @pl.when(q < N_ROW_CH)
            def _():
                r0 = q * ROW_CH
                pltpu.sync_copy(
                    acc_sh.at[pl.ds(r0, ROW_CH)], out_hbm.at[c, pl.ds(r0, ROW_CH)]
                )

    return k(src_r, dst_r, h)


BR = 1000


def _dis(deg_ref):
    return lax.rsqrt(deg_ref[:, 0:1] + deg_ref[:, 1:2] + 1.0)


def _tc1_body(x_ref, w_ref, deg_ref, out_ref):
    mm = jnp.dot(x_ref[...], w_ref[...], preferred_element_type=jnp.float32)
    out_ref[...] = mm * _dis(deg_ref)


def _tc1(x, W0, degT):
    return pl.pallas_call(
        _tc1_body,
        grid=(N // BR,),
        in_specs=[
            pl.BlockSpec((BR, D), lambda i: (i, 0)),
            pl.BlockSpec((D, D), lambda i: (0, 0)),
            pl.BlockSpec((BR, 2), lambda i: (i, 0)),
        ],
        out_specs=pl.BlockSpec((BR, D), lambda i: (i, 0)),
        out_shape=jax.ShapeDtypeStruct((N, D), jnp.float32),
    )(x, W0, degT)


def _tc2_body(p_ref, h_ref, deg_ref, b0_ref, g_ref, be_ref, w1_ref, out_ref):
    dis = _dis(deg_ref)
    t = (p_ref[0] + p_ref[1] - h_ref[...]) * dis + b0_ref[...]
    mu = jnp.mean(t, axis=1, keepdims=True)
    var = jnp.mean((t - mu) ** 2, axis=1, keepdims=True)
    tn = (t - mu) * lax.rsqrt(var + 1e-5) * g_ref[...] + be_ref[...]
    tr = jnp.maximum(tn, 0.0)
    out_ref[...] = (
        jnp.dot(tr, w1_ref[...], preferred_element_type=jnp.float32) * dis
    )


def _tc2(p, h0p, degT, b0, gamma, beta, W1):
    return pl.pallas_call(
        _tc2_body,
        grid=(N // BR,),
        in_specs=[
            pl.BlockSpec((NC, BR, D), lambda i: (0, i, 0)),
            pl.BlockSpec((BR, D), lambda i: (i, 0)),
            pl.BlockSpec((BR, 2), lambda i: (i, 0)),
            pl.BlockSpec((1, D), lambda i: (0, 0)),
            pl.BlockSpec((1, D), lambda i: (0, 0)),
            pl.BlockSpec((1, D), lambda i: (0, 0)),
            pl.BlockSpec((D, D), lambda i: (0, 0)),
        ],
        out_specs=pl.BlockSpec((BR, D), lambda i: (i, 0)),
        out_shape=jax.ShapeDtypeStruct((N, D), jnp.float32),
    )(p, h0p, degT, b0, gamma, beta, W1)


def _tc3_body(p_ref, h_ref, deg_ref, b1_ref, out_ref):
    out_ref[...] = (p_ref[0] + p_ref[1] - h_ref[...]) * _dis(deg_ref) + b1_ref[...]


def _tc3(p, h1p, degT, b1):
    return pl.pallas_call(
        _tc3_body,
        grid=(N // BR,),
        in_specs=[
            pl.BlockSpec((NC, BR, D), lambda i: (0, i, 0)),
            pl.BlockSpec((BR, D), lambda i: (i, 0)),
            pl.BlockSpec((BR, 2), lambda i: (i, 0)),
            pl.BlockSpec((1, D), lambda i: (0, 0)),
        ],
        out_specs=pl.BlockSpec((BR, D), lambda i: (i, 0)),
        out_shape=jax.ShapeDtypeStruct((N, D), jnp.float32),
    )(p, h1p, degT, b1)


def kernel(x, edge_index, W0, b0, gamma, beta, W1, b1):
    ei = edge_index.astype(jnp.int32)
    src_r = ei[0].reshape(NW, CH, C)
    dst_r = ei[1].reshape(NW, CH, C)
    dst_r4 = ei[1].reshape(NW, CH, 1, C)
    b0r = b0.reshape(1, D)
    b1r = b1.reshape(1, D)
    gr = gamma.reshape(1, D)
    ber = beta.reshape(1, D)

    degf = _sc_deg(dst_r)
    degT = jnp.stack([degf[:N], degf[NP_DEG : NP_DEG + N]], axis=1)
    h0p = _tc1(x, W0, degT)
    p0 = _sc_msg(src_r, dst_r4, h0p)
    h1p = _tc2(p0, h0p, degT, b0r, gr, ber, W1)
    p1 = _sc_msg(src_r, dst_r4, h1p)
    out = _tc3(p1, h1p, degT, b1r)
    return out


# speedup vs baseline: 35.5226x; 1.0015x over previous
import functools

import jax
import jax.numpy as jnp
from jax import lax
from jax.experimental import pallas as pl
from jax.experimental.pallas import tpu as pltpu
from jax.experimental.pallas import tpu_sc as plsc

N = 10000
D = 128
E = 320000
NC = 2
NS = 16
NW = NC * NS
EPW = E // NW
C = 80
CH = EPW // C

NP_DEG = 10240
DEG_CH = NP_DEG // NS

ROW_CH = 200
N_ROW_CH = N // ROW_CH
ROUNDS = 4


def _sc_mesh():
    return plsc.VectorSubcoreMesh(core_axis_name="c", subcore_axis_name="s")


def _sc_deg(dst_r):
    @functools.partial(
        pl.kernel,
        mesh=_sc_mesh(),
        out_type=jax.ShapeDtypeStruct((NC * NP_DEG,), jnp.float32),
        scratch_types=[
            pltpu.VMEM((CH, C), jnp.int32),
            pltpu.VMEM((C,), jnp.float32),
            pltpu.VMEM((DEG_CH,), jnp.float32),
            pltpu.VMEM_SHARED((NP_DEG,), jnp.float32),
        ],
    )
    def k(dst_hbm, out_hbm, idx_v, ones_v, zer_v, acc_sh):
        c = lax.axis_index("c")
        s = lax.axis_index("s")
        w = c * NS + s
        for i in range(C // 16):
            ones_v[pl.ds(i * 16, 16)] = jnp.ones((16,), jnp.float32)
        for i in range(DEG_CH // 16):
            zer_v[pl.ds(i * 16, 16)] = jnp.zeros((16,), jnp.float32)
        pltpu.sync_copy(zer_v, acc_sh.at[pl.ds(s * DEG_CH, DEG_CH)])
        pltpu.sync_copy(dst_hbm.at[w], idx_v)
        plsc.subcore_barrier()

        def step(i, carry):
            pltpu.sync_copy(ones_v, acc_sh.at[idx_v.at[i]], add=True)
            return carry

        lax.fori_loop(0, CH, step, 0)
        plsc.subcore_barrier()
        pltpu.sync_copy(
            acc_sh.at[pl.ds(s * DEG_CH, DEG_CH)],
            out_hbm.at[pl.ds(c * NP_DEG + s * DEG_CH, DEG_CH)],
        )

    return k(dst_r)


def _sc_msg(src_r, dst_r, h):
    @functools.partial(
        pl.kernel,
        mesh=_sc_mesh(),
        out_type=jax.ShapeDtypeStruct((NC, N, D), jnp.float32),
        scratch_types=[
            pltpu.VMEM((CH, C), jnp.int32),
            pltpu.VMEM((C,), jnp.int32),
            pltpu.VMEM((C,), jnp.int32),
            pltpu.VMEM((C,), jnp.int32),
            pltpu.VMEM((C, D), jnp.float32),
            pltpu.VMEM((C, D), jnp.float32),
            pltpu.VMEM((C, D), jnp.float32),
            pltpu.VMEM_SHARED((N, D), jnp.float32),
            pltpu.SemaphoreType.DMA,
            pltpu.SemaphoreType.DMA,
            pltpu.SemaphoreType.DMA,
            pltpu.SemaphoreType.DMA,
            pltpu.SemaphoreType.DMA,
            pltpu.SemaphoreType.DMA,
            pltpu.SemaphoreType.DMA,
            pltpu.SemaphoreType.DMA,
            pltpu.SemaphoreType.DMA,
        ],
    )
    def k(src_hbm, dst_hbm, h_hbm, out_hbm, src_v, dst_c0, dst_c1, dst_c2,
          rows0, rows1, rows2, acc_sh,
          semi0, semi1, semi2, semg0, semg1, semg2, sems0, sems1, sems2):
        c = lax.axis_index("c")
        s = lax.axis_index("s")
        w = c * NS + s
        for j in range(ROUNDS):
            q = s + j * NS

            @pl.when(q < N_ROW_CH)
            def _():
                r0 = q * ROW_CH
                pltpu.sync_copy(
                    h_hbm.at[pl.ds(r0, ROW_CH)], acc_sh.at[pl.ds(r0, ROW_CH)]
                )

        pltpu.sync_copy(src_hbm.at[w], src_v)
        plsc.subcore_barrier()

        bufs = (
            (dst_c0, rows0, semi0, semg0, sems0),
            (dst_c1, rows1, semi1, semg1, sems1),
            (dst_c2, rows2, semi2, semg2, sems2),
        )

        def _start(i, b):
            dstb, rowsb, semi, semg, _ = bufs[b]
            pltpu.async_copy(dst_hbm.at[w, i, 0], dstb, semi)
            pltpu.async_copy(h_hbm.at[src_v.at[i]], rowsb, semg)

        def _wait_scat(b):
            dstb, rowsb, _, _, sems = bufs[b]
            pltpu.make_async_copy(rowsb, acc_sh.at[dstb], sems).wait()

        def _step(i, b, first, last):
            dstb, rowsb, semi, semg, sems = bufs[b]
            if not first:

                @pl.when(i >= 1)
                def _():
                    _wait_scat((b + 2) % 3)

            if not last:

                @pl.when(i + 2 < CH)
                def _():
                    _start(i + 2, (b + 2) % 3)

            pltpu.make_async_copy(dst_hbm.at[w, i, 0], dstb, semi).wait()
            pltpu.make_async_copy(h_hbm.at[src_v.at[i]], rowsb, semg).wait()
            pltpu.async_copy(rowsb, acc_sh.at[dstb], sems, add=True)

        _start(0, 0)
        _start(1, 1)
        _step(0, 0, True, False)

        def step(j, carry):
            i0 = 3 * j + 1
            _step(i0, 1, False, False)
            _step(i0 + 1, 2, False, False)
            _step(i0 + 2, 0, False, False)
            return carry

        lax.fori_loop(0, (CH - 2) // 3, step, 0)
        _step(CH - 1, (CH - 1) % 3, False, True)
        _wait_scat((CH - 1) % 3)
        plsc.subcore_barrier()
        for j in range(ROUNDS):
            q = s + j * NS

            @pl.when(q < N_ROW_CH)
            def _():
                r0 = q * ROW_CH
                pltpu.sync_copy(
                    acc_sh.at[pl.ds(r0, ROW_CH)], out_hbm.at[c, pl.ds(r0, ROW_CH)]
                )

    return k(src_r, dst_r, h)


BR = 1000


def _dis(deg_ref):
    return lax.rsqrt(deg_ref[:, 0:1] + deg_ref[:, 1:2] + 1.0)


def _tc1_body(x_ref, w_ref, deg_ref, out_ref):
    mm = jnp.dot(x_ref[...], w_ref[...], preferred_element_type=jnp.float32)
    out_ref[...] = mm * _dis(deg_ref)


def _tc1(x, W0, degT):
    return pl.pallas_call(
        _tc1_body,
        grid=(N // BR,),
        in_specs=[
            pl.BlockSpec((BR, D), lambda i: (i, 0)),
            pl.BlockSpec((D, D), lambda i: (0, 0)),
            pl.BlockSpec((BR, 2), lambda i: (i, 0)),
        ],
        out_specs=pl.BlockSpec((BR, D), lambda i: (i, 0)),
        out_shape=jax.ShapeDtypeStruct((N, D), jnp.float32),
    )(x, W0, degT)


def _tc2_body(p_ref, h_ref, deg_ref, b0_ref, g_ref, be_ref, w1_ref, out_ref):
    dis = _dis(deg_ref)
    t = (p_ref[0] + p_ref[1] - h_ref[...]) * dis + b0_ref[...]
    mu = jnp.mean(t, axis=1, keepdims=True)
    var = jnp.mean((t - mu) ** 2, axis=1, keepdims=True)
    tn = (t - mu) * lax.rsqrt(var + 1e-5) * g_ref[...] + be_ref[...]
    tr = jnp.maximum(tn, 0.0)
    out_ref[...] = (
        jnp.dot(tr, w1_ref[...], preferred_element_type=jnp.float32) * dis
    )


def _tc2(p, h0p, degT, b0, gamma, beta, W1):
    return pl.pallas_call(
        _tc2_body,
        grid=(N // BR,),
        in_specs=[
            pl.BlockSpec((NC, BR, D), lambda i: (0, i, 0)),
            pl.BlockSpec((BR, D), lambda i: (i, 0)),
            pl.BlockSpec((BR, 2), lambda i: (i, 0)),
            pl.BlockSpec((1, D), lambda i: (0, 0)),
            pl.BlockSpec((1, D), lambda i: (0, 0)),
            pl.BlockSpec((1, D), lambda i: (0, 0)),
            pl.BlockSpec((D, D), lambda i: (0, 0)),
        ],
        out_specs=pl.BlockSpec((BR, D), lambda i: (i, 0)),
        out_shape=jax.ShapeDtypeStruct((N, D), jnp.float32),
    )(p, h0p, degT, b0, gamma, beta, W1)


def _tc3_body(p_ref, h_ref, deg_ref, b1_ref, out_ref):
    out_ref[...] = (p_ref[0] + p_ref[1] - h_ref[...]) * _dis(deg_ref) + b1_ref[...]


def _tc3(p, h1p, degT, b1):
    return pl.pallas_call(
        _tc3_body,
        grid=(N // BR,),
        in_specs=[
            pl.BlockSpec((NC, BR, D), lambda i: (0, i, 0)),
            pl.BlockSpec((BR, D), lambda i: (i, 0)),
            pl.BlockSpec((BR, 2), lambda i: (i, 0)),
            pl.BlockSpec((1, D), lambda i: (0, 0)),
        ],
        out_specs=pl.BlockSpec((BR, D), lambda i: (i, 0)),
        out_shape=jax.ShapeDtypeStruct((N, D), jnp.float32),
    )(p, h1p, degT, b1)


def kernel(x, edge_index, W0, b0, gamma, beta, W1, b1):
    ei = edge_index.astype(jnp.int32)
    src_r = ei[0].reshape(NW, CH, C)
    dst_r = ei[1].reshape(NW, CH, C)
    dst_r4 = ei[1].reshape(NW, CH, 1, C)
    b0r = b0.reshape(1, D)
    b1r = b1.reshape(1, D)
    gr = gamma.reshape(1, D)
    ber = beta.reshape(1, D)

    degf = _sc_deg(dst_r)
    degT = jnp.stack([degf[:N], degf[NP_DEG : NP_DEG + N]], axis=1)
    h0p = _tc1(x, W0, degT)
    p0 = _sc_msg(src_r, dst_r4, h0p)
    h1p = _tc2(p0, h0p, degT, b0r, gr, ber, W1)
    p1 = _sc_msg(src_r, dst_r4, h1p)
    out = _tc3(p1, h1p, degT, b1r)
    return out


# speedup vs baseline: 36.4625x; 1.0265x over previous
import functools

import jax
import jax.numpy as jnp
from jax import lax
from jax.experimental import pallas as pl
from jax.experimental.pallas import tpu as pltpu
from jax.experimental.pallas import tpu_sc as plsc

N = 10000
D = 128
E = 320000
NC = 2
NS = 16
NW = NC * NS
EPW = E // NW
C = 80
CH = EPW // C

NP_DEG = 10240
DEG_CH = NP_DEG // NS

ROW_CH = 200
N_ROW_CH = N // ROW_CH
ROUNDS = 4


def _sc_mesh():
    return plsc.VectorSubcoreMesh(core_axis_name="c", subcore_axis_name="s")


def _sc_deg(dst_r):
    @functools.partial(
        pl.kernel,
        mesh=_sc_mesh(),
        out_type=jax.ShapeDtypeStruct((NC * NP_DEG,), jnp.float32),
        scratch_types=[
            pltpu.VMEM((CH, C), jnp.int32),
            pltpu.VMEM((C,), jnp.float32),
            pltpu.VMEM((DEG_CH,), jnp.float32),
            pltpu.VMEM_SHARED((NP_DEG,), jnp.float32),
        ],
    )
    def k(dst_hbm, out_hbm, idx_v, ones_v, zer_v, acc_sh):
        c = lax.axis_index("c")
        s = lax.axis_index("s")
        w = c * NS + s
        for i in range(C // 16):
            ones_v[pl.ds(i * 16, 16)] = jnp.ones((16,), jnp.float32)
        for i in range(DEG_CH // 16):
            zer_v[pl.ds(i * 16, 16)] = jnp.zeros((16,), jnp.float32)
        pltpu.sync_copy(zer_v, acc_sh.at[pl.ds(s * DEG_CH, DEG_CH)])
        pltpu.sync_copy(dst_hbm.at[w], idx_v)
        plsc.subcore_barrier()

        def step(i, carry):
            pltpu.sync_copy(ones_v, acc_sh.at[idx_v.at[i]], add=True)
            return carry

        lax.fori_loop(0, CH, step, 0)
        plsc.subcore_barrier()
        pltpu.sync_copy(
            acc_sh.at[pl.ds(s * DEG_CH, DEG_CH)],
            out_hbm.at[pl.ds(c * NP_DEG + s * DEG_CH, DEG_CH)],
        )

    return k(dst_r)


def _sc_msg(src_r, dst_r, h):
    @functools.partial(
        pl.kernel,
        mesh=_sc_mesh(),
        out_type=jax.ShapeDtypeStruct((NC, N, D), jnp.float32),
        scratch_types=[
            pltpu.VMEM((CH, C), jnp.int32),
            pltpu.VMEM((C,), jnp.int32),
            pltpu.VMEM((C,), jnp.int32),
            pltpu.VMEM((C,), jnp.int32),
            pltpu.VMEM((C, D), jnp.float32),
            pltpu.VMEM((C, D), jnp.float32),
            pltpu.VMEM((C, D), jnp.float32),
            pltpu.VMEM_SHARED((N, D), jnp.float32),
            pltpu.SemaphoreType.DMA,
            pltpu.SemaphoreType.DMA,
            pltpu.SemaphoreType.DMA,
            pltpu.SemaphoreType.DMA,
            pltpu.SemaphoreType.DMA,
            pltpu.SemaphoreType.DMA,
            pltpu.SemaphoreType.DMA,
            pltpu.SemaphoreType.DMA,
            pltpu.SemaphoreType.DMA,
        ],
    )
    def k(src_hbm, dst_hbm, h_hbm, out_hbm, src_v, dst_c0, dst_c1, dst_c2,
          rows0, rows1, rows2, acc_sh,
          semi0, semi1, semi2, semg0, semg1, semg2, sems0, sems1, sems2):
        c = lax.axis_index("c")
        s = lax.axis_index("s")
        w = c * NS + s
        for j in range(ROUNDS):
            q = s + j * NS

            @pl.when(q < N_ROW_CH)
            def _():
                r0 = q * ROW_CH
                pltpu.sync_copy(
                    h_hbm.at[pl.ds(r0, ROW_CH)], acc_sh.at[pl.ds(r0, ROW_CH)]
                )

        pltpu.sync_copy(src_hbm.at[w], src_v)
        plsc.subcore_barrier()

        bufs = (
            (dst_c0, rows0, semi0, semg0, sems0),
            (dst_c1, rows1, semi1, semg1, sems1),
            (dst_c2, rows2, semi2, semg2, sems2),
        )

        def _start(i, b):
            dstb, rowsb, semi, semg, _ = bufs[b]
            pltpu.async_copy(dst_hbm.at[w, i, 0], dstb, semi)
            pltpu.async_copy(h_hbm.at[src_v.at[i]], rowsb, semg)

        def _wait_scat(b):
            dstb, rowsb, _, _, sems = bufs[b]
            pltpu.make_async_copy(rowsb, acc_sh.at[dstb], sems).wait()

        def _step(i, b, first, last):
            dstb, rowsb, semi, semg, sems = bufs[b]
            if not first:

                @pl.when(i >= 1)
                def _():
                    _wait_scat((b + 2) % 3)

            if not last:

                @pl.when(i + 2 < CH)
                def _():
                    _start(i + 2, (b + 2) % 3)

            pltpu.make_async_copy(dst_hbm.at[w, i, 0], dstb, semi).wait()
            pltpu.make_async_copy(h_hbm.at[src_v.at[i]], rowsb, semg).wait()
            pltpu.async_copy(rowsb, acc_sh.at[dstb], sems, add=True)

        _start(0, 0)
        _start(1, 1)
        _step(0, 0, True, False)

        def step(j, carry):
            i0 = 3 * j + 1
            _step(i0, 1, False, False)
            _step(i0 + 1, 2, False, False)
            _step(i0 + 2, 0, False, False)
            return carry

        lax.fori_loop(0, (CH - 2) // 3, step, 0)
        _step(CH - 1, (CH - 1) % 3, False, True)
        _wait_scat((CH - 1) % 3)
        plsc.subcore_barrier()
        for j in range(ROUNDS):
            q = s + j * NS

            @pl.when(q < N_ROW_CH)
            def _():
                r0 = q * ROW_CH
                pltpu.sync_copy(
                    acc_sh.at[pl.ds(r0, ROW_CH)], out_hbm.at[c, pl.ds(r0, ROW_CH)]
                )

    return k(src_r, dst_r, h)


BR = 2000


def _dis(deg_ref):
    return lax.rsqrt(deg_ref[:, 0:1] + deg_ref[:, 1:2] + 1.0)


def _tc1_body(x_ref, w_ref, deg_ref, out_ref):
    mm = jnp.dot(x_ref[...], w_ref[...], preferred_element_type=jnp.float32)
    out_ref[...] = mm * _dis(deg_ref)


def _tc1(x, W0, degT):
    return pl.pallas_call(
        _tc1_body,
        grid=(N // BR,),
        in_specs=[
            pl.BlockSpec((BR, D), lambda i: (i, 0)),
            pl.BlockSpec((D, D), lambda i: (0, 0)),
            pl.BlockSpec((BR, 2), lambda i: (i, 0)),
        ],
        out_specs=pl.BlockSpec((BR, D), lambda i: (i, 0)),
        out_shape=jax.ShapeDtypeStruct((N, D), jnp.float32),
    )(x, W0, degT)


def _tc2_body(p_ref, h_ref, deg_ref, b0_ref, g_ref, be_ref, w1_ref, out_ref):
    dis = _dis(deg_ref)
    t = (p_ref[0] + p_ref[1] - h_ref[...]) * dis + b0_ref[...]
    mu = jnp.mean(t, axis=1, keepdims=True)
    var = jnp.mean((t - mu) ** 2, axis=1, keepdims=True)
    tn = (t - mu) * lax.rsqrt(var + 1e-5) * g_ref[...] + be_ref[...]
    tr = jnp.maximum(tn, 0.0)
    out_ref[...] = (
        jnp.dot(tr, w1_ref[...], preferred_element_type=jnp.float32) * dis
    )


def _tc2(p, h0p, degT, b0, gamma, beta, W1):
    return pl.pallas_call(
        _tc2_body,
        grid=(N // BR,),
        in_specs=[
            pl.BlockSpec((NC, BR, D), lambda i: (0, i, 0)),
            pl.BlockSpec((BR, D), lambda i: (i, 0)),
            pl.BlockSpec((BR, 2), lambda i: (i, 0)),
            pl.BlockSpec((1, D), lambda i: (0, 0)),
            pl.BlockSpec((1, D), lambda i: (0, 0)),
            pl.BlockSpec((1, D), lambda i: (0, 0)),
            pl.BlockSpec((D, D), lambda i: (0, 0)),
        ],
        out_specs=pl.BlockSpec((BR, D), lambda i: (i, 0)),
        out_shape=jax.ShapeDtypeStruct((N, D), jnp.float32),
    )(p, h0p, degT, b0, gamma, beta, W1)


def _tc3_body(p_ref, h_ref, deg_ref, b1_ref, out_ref):
    out_ref[...] = (p_ref[0] + p_ref[1] - h_ref[...]) * _dis(deg_ref) + b1_ref[...]


def _tc3(p, h1p, degT, b1):
    return pl.pallas_call(
        _tc3_body,
        grid=(N // BR,),
        in_specs=[
            pl.BlockSpec((NC, BR, D), lambda i: (0, i, 0)),
            pl.BlockSpec((BR, D), lambda i: (i, 0)),
            pl.BlockSpec((BR, 2), lambda i: (i, 0)),
            pl.BlockSpec((1, D), lambda i: (0, 0)),
        ],
        out_specs=pl.BlockSpec((BR, D), lambda i: (i, 0)),
        out_shape=jax.ShapeDtypeStruct((N, D), jnp.float32),
    )(p, h1p, degT, b1)


def kernel(x, edge_index, W0, b0, gamma, beta, W1, b1):
    ei = edge_index.astype(jnp.int32)
    src_r = ei[0].reshape(NW, CH, C)
    dst_r = ei[1].reshape(NW, CH, C)
    dst_r4 = ei[1].reshape(NW, CH, 1, C)
    b0r = b0.reshape(1, D)
    b1r = b1.reshape(1, D)
    gr = gamma.reshape(1, D)
    ber = beta.reshape(1, D)

    degf = _sc_deg(dst_r)
    degT = jnp.stack([degf[:N], degf[NP_DEG : NP_DEG + N]], axis=1)
    h0p = _tc1(x, W0, degT)
    p0 = _sc_msg(src_r, dst_r4, h0p)
    h1p = _tc2(p0, h0p, degT, b0r, gr, ber, W1)
    p1 = _sc_msg(src_r, dst_r4, h1p)
    out = _tc3(p1, h1p, degT, b1r)
    return out


# speedup vs baseline: 37.1860x; 1.0198x over previous
import functools

import jax
import jax.numpy as jnp
from jax import lax
from jax.experimental import pallas as pl
from jax.experimental.pallas import tpu as pltpu
from jax.experimental.pallas import tpu_sc as plsc

N = 10000
D = 128
E = 320000
NC = 2
NS = 16
NW = NC * NS
EPW = E // NW
C = 80
CH = EPW // C

NP_DEG = 10240
DEG_CH = NP_DEG // NS

ROW_CH = 200
N_ROW_CH = N // ROW_CH
ROUNDS = 4


def _sc_mesh():
    return plsc.VectorSubcoreMesh(core_axis_name="c", subcore_axis_name="s")


def _sc_deg(dst_r):
    @functools.partial(
        pl.kernel,
        mesh=_sc_mesh(),
        out_type=jax.ShapeDtypeStruct((NC * NP_DEG,), jnp.float32),
        scratch_types=[
            pltpu.VMEM((CH, C), jnp.int32),
            pltpu.VMEM((C,), jnp.float32),
            pltpu.VMEM((DEG_CH,), jnp.float32),
            pltpu.VMEM_SHARED((NP_DEG,), jnp.float32),
            pltpu.SemaphoreType.DMA,
        ],
    )
    def k(dst_hbm, out_hbm, idx_v, ones_v, zer_v, acc_sh, sem):
        c = lax.axis_index("c")
        s = lax.axis_index("s")
        w = c * NS + s
        for i in range(C // 16):
            ones_v[pl.ds(i * 16, 16)] = jnp.ones((16,), jnp.float32)
        for i in range(DEG_CH // 16):
            zer_v[pl.ds(i * 16, 16)] = jnp.zeros((16,), jnp.float32)
        pltpu.sync_copy(zer_v, acc_sh.at[pl.ds(s * DEG_CH, DEG_CH)])
        pltpu.sync_copy(dst_hbm.at[w], idx_v)
        plsc.subcore_barrier()

        def step(i, carry):
            pltpu.async_copy(ones_v, acc_sh.at[idx_v.at[i]], sem, add=True)
            return carry

        lax.fori_loop(0, CH, step, 0)

        def drain(i, carry):
            pltpu.make_async_copy(ones_v, acc_sh.at[idx_v.at[i]], sem).wait()
            return carry

        lax.fori_loop(0, CH, drain, 0)
        plsc.subcore_barrier()
        pltpu.sync_copy(
            acc_sh.at[pl.ds(s * DEG_CH, DEG_CH)],
            out_hbm.at[pl.ds(c * NP_DEG + s * DEG_CH, DEG_CH)],
        )

    return k(dst_r)


def _sc_msg(src_r, dst_r, h):
    @functools.partial(
        pl.kernel,
        mesh=_sc_mesh(),
        out_type=jax.ShapeDtypeStruct((NC, N, D), jnp.float32),
        scratch_types=[
            pltpu.VMEM((CH, C), jnp.int32),
            pltpu.VMEM((C,), jnp.int32),
            pltpu.VMEM((C,), jnp.int32),
            pltpu.VMEM((C,), jnp.int32),
            pltpu.VMEM((C, D), jnp.float32),
            pltpu.VMEM((C, D), jnp.float32),
            pltpu.VMEM((C, D), jnp.float32),
            pltpu.VMEM_SHARED((N, D), jnp.float32),
            pltpu.SemaphoreType.DMA,
            pltpu.SemaphoreType.DMA,
            pltpu.SemaphoreType.DMA,
            pltpu.SemaphoreType.DMA,
            pltpu.SemaphoreType.DMA,
            pltpu.SemaphoreType.DMA,
            pltpu.SemaphoreType.DMA,
            pltpu.SemaphoreType.DMA,
            pltpu.SemaphoreType.DMA,
        ],
    )
    def k(src_hbm, dst_hbm, h_hbm, out_hbm, src_v, dst_c0, dst_c1, dst_c2,
          rows0, rows1, rows2, acc_sh,
          semi0, semi1, semi2, semg0, semg1, semg2, sems0, sems1, sems2):
        c = lax.axis_index("c")
        s = lax.axis_index("s")
        w = c * NS + s
        for j in range(ROUNDS):
            q = s + j * NS

            @pl.when(q < N_ROW_CH)
            def _():
                r0 = q * ROW_CH
                pltpu.sync_copy(
                    h_hbm.at[pl.ds(r0, ROW_CH)], acc_sh.at[pl.ds(r0, ROW_CH)]
                )

        pltpu.sync_copy(src_hbm.at[w], src_v)
        plsc.subcore_barrier()

        bufs = (
            (dst_c0, rows0, semi0, semg0, sems0),
            (dst_c1, rows1, semi1, semg1, sems1),
            (dst_c2, rows2, semi2, semg2, sems2),
        )

        def _start(i, b):
            dstb, rowsb, semi, semg, _ = bufs[b]
            pltpu.async_copy(dst_hbm.at[w, i, 0], dstb, semi)
            pltpu.async_copy(h_hbm.at[src_v.at[i]], rowsb, semg)

        def _wait_scat(b):
            dstb, rowsb, _, _, sems = bufs[b]
            pltpu.make_async_copy(rowsb, acc_sh.at[dstb], sems).wait()

        def _step(i, b, first, last):
            dstb, rowsb, semi, semg, sems = bufs[b]
            if not first:

                @pl.when(i >= 1)
                def _():
                    _wait_scat((b + 2) % 3)

            if not last:

                @pl.when(i + 2 < CH)
                def _():
                    _start(i + 2, (b + 2) % 3)

            pltpu.make_async_copy(dst_hbm.at[w, i, 0], dstb, semi).wait()
            pltpu.make_async_copy(h_hbm.at[src_v.at[i]], rowsb, semg).wait()
            pltpu.async_copy(rowsb, acc_sh.at[dstb], sems, add=True)

        _start(0, 0)
        _start(1, 1)
        _step(0, 0, True, False)

        def step(j, carry):
            i0 = 3 * j + 1
            _step(i0, 1, False, False)
            _step(i0 + 1, 2, False, False)
            _step(i0 + 2, 0, False, False)
            return carry

        lax.fori_loop(0, (CH - 2) // 3, step, 0)
        _step(CH - 1, (CH - 1) % 3, False, True)
        _wait_scat((CH - 1) % 3)
        plsc.subcore_barrier()
        for j in range(ROUNDS):
            q = s + j * NS

            @pl.when(q < N_ROW_CH)
            def _():
                r0 = q * ROW_CH
                pltpu.sync_copy(
                    acc_sh.at[pl.ds(r0, ROW_CH)], out_hbm.at[c, pl.ds(r0, ROW_CH)]
                )

    return k(src_r, dst_r, h)


BR = 2000


def _dis(deg_ref):
    return lax.rsqrt(deg_ref[:, 0:1] + deg_ref[:, 1:2] + 1.0)


def _tc1_body(x_ref, w_ref, deg_ref, out_ref):
    mm = jnp.dot(x_ref[...], w_ref[...], preferred_element_type=jnp.float32)
    out_ref[...] = mm * _dis(deg_ref)


def _tc1(x, W0, degT):
    return pl.pallas_call(
        _tc1_body,
        grid=(N // BR,),
        in_specs=[
            pl.BlockSpec((BR, D), lambda i: (i, 0)),
            pl.BlockSpec((D, D), lambda i: (0, 0)),
            pl.BlockSpec((BR, 2), lambda i: (i, 0)),
        ],
        out_specs=pl.BlockSpec((BR, D), lambda i: (i, 0)),
        out_shape=jax.ShapeDtypeStruct((N, D), jnp.float32),
    )(x, W0, degT)


def _tc2_body(p_ref, h_ref, deg_ref, b0_ref, g_ref, be_ref, w1_ref, out_ref):
    dis = _dis(deg_ref)
    t = (p_ref[0] + p_ref[1] - h_ref[...]) * dis + b0_ref[...]
    mu = jnp.mean(t, axis=1, keepdims=True)
    var = jnp.mean((t - mu) ** 2, axis=1, keepdims=True)
    tn = (t - mu) * lax.rsqrt(var + 1e-5) * g_ref[...] + be_ref[...]
    tr = jnp.maximum(tn, 0.0)
    out_ref[...] = (
        jnp.dot(tr, w1_ref[...], preferred_element_type=jnp.float32) * dis
    )


def _tc2(p, h0p, degT, b0, gamma, beta, W1):
    return pl.pallas_call(
        _tc2_body,
        grid=(N // BR,),
        in_specs=[
            pl.BlockSpec((NC, BR, D), lambda i: (0, i, 0)),
            pl.BlockSpec((BR, D), lambda i: (i, 0)),
            pl.BlockSpec((BR, 2), lambda i: (i, 0)),
            pl.BlockSpec((1, D), lambda i: (0, 0)),
            pl.BlockSpec((1, D), lambda i: (0, 0)),
            pl.BlockSpec((1, D), lambda i: (0, 0)),
            pl.BlockSpec((D, D), lambda i: (0, 0)),
        ],
        out_specs=pl.BlockSpec((BR, D), lambda i: (i, 0)),
        out_shape=jax.ShapeDtypeStruct((N, D), jnp.float32),
    )(p, h0p, degT, b0, gamma, beta, W1)


def _tc3_body(p_ref, h_ref, deg_ref, b1_ref, out_ref):
    out_ref[...] = (p_ref[0] + p_ref[1] - h_ref[...]) * _dis(deg_ref) + b1_ref[...]


def _tc3(p, h1p, degT, b1):
    return pl.pallas_call(
        _tc3_body,
        grid=(N // BR,),
        in_specs=[
            pl.BlockSpec((NC, BR, D), lambda i: (0, i, 0)),
            pl.BlockSpec((BR, D), lambda i: (i, 0)),
            pl.BlockSpec((BR, 2), lambda i: (i, 0)),
            pl.BlockSpec((1, D), lambda i: (0, 0)),
        ],
        out_specs=pl.BlockSpec((BR, D), lambda i: (i, 0)),
        out_shape=jax.ShapeDtypeStruct((N, D), jnp.float32),
    )(p, h1p, degT, b1)


def kernel(x, edge_index, W0, b0, gamma, beta, W1, b1):
    ei = edge_index.astype(jnp.int32)
    src_r = ei[0].reshape(NW, CH, C)
    dst_r = ei[1].reshape(NW, CH, C)
    dst_r4 = ei[1].reshape(NW, CH, 1, C)
    b0r = b0.reshape(1, D)
    b1r = b1.reshape(1, D)
    gr = gamma.reshape(1, D)
    ber = beta.reshape(1, D)

    degf = _sc_deg(dst_r)
    degT = jnp.stack([degf[:N], degf[NP_DEG : NP_DEG + N]], axis=1)
    h0p = _tc1(x, W0, degT)
    p0 = _sc_msg(src_r, dst_r4, h0p)
    h1p = _tc2(p0, h0p, degT, b0r, gr, ber, W1)
    p1 = _sc_msg(src_r, dst_r4, h1p)
    out = _tc3(p1, h1p, degT, b1r)
    return out
